# baseline (speedup 1.0000x reference)
import functools

import jax
import jax.numpy as jnp
from jax import lax
from jax.experimental import pallas as pl
from jax.experimental.pallas import tpu as pltpu
from jax.experimental.pallas import tpu_sc as plsc

_N = 10000
_NPAD = 10240
_E = 160000
_C = 128
_CHUNK = 128
_NCHUNKS = 1280
_EPAD = _NCHUNKS * _CHUNK
_NC, _NS = 2, 16
_NW = _NC * _NS
_CPW = _NCHUNKS // _NW
_ECH = 64
_NCH2 = _EPAD // _ECH
_CPT = _NCH2 // _NW
_RPT = _NPAD // _NS
_NREL = 3

_MESH = plsc.VectorSubcoreMesh(
    core_axis_name="c", subcore_axis_name="s", num_cores=_NC, num_subcores=_NS)


def _sc_prep_body(src_hbm, et_hbm, dst_hbm, gidx_hbm, degp_hbm,
                  src_all, et_all, dst_all, g_all, ones_v, zv, dacc, ssem):
    cid = lax.axis_index("c")
    sid = lax.axis_index("s")
    wid = cid * _NS + sid

    pltpu.sync_copy(src_hbm.at[pl.ds(wid * _CPW, _CPW)], src_all)
    pltpu.sync_copy(et_hbm.at[pl.ds(wid * _CPW, _CPW)], et_all)
    pltpu.sync_copy(dst_hbm.at[pl.ds(wid * _CPW, _CPW)], dst_all)

    def zstep(k, carry):
        zv[pl.ds(k * 16, 16)] = jnp.zeros((16,), jnp.float32)
        return carry
    lax.fori_loop(0, _RPT // 16, zstep, 0)
    for k in range(_CHUNK // 16):
        ones_v[pl.ds(k * 16, 16)] = jnp.ones((16,), jnp.float32)
    pltpu.sync_copy(zv, dacc.at[pl.ds(sid * _RPT, _RPT)])
    plsc.subcore_barrier()

    def step(t, carry):
        for k in range(_CHUNK // 16):
            sl = pl.ds(k * 16, 16)
            g_all[t, sl] = et_all[t, sl] * _N + src_all[t, sl]
        pltpu.async_copy(ones_v, dacc.at[dst_all.at[t]], ssem, add=True)
        return carry
    lax.fori_loop(0, _CPW, step, 0)
    pltpu.sync_copy(g_all, gidx_hbm.at[pl.ds(wid * _CPW, _CPW)])

    def drain(t, carry):
        pltpu.make_async_copy(ones_v, dacc.at[dst_all.at[t]], ssem).wait()
        return carry
    lax.fori_loop(0, _CPW, drain, 0)
    plsc.subcore_barrier()
    pltpu.sync_copy(dacc.at[pl.ds(sid * _RPT, _RPT)],
                    degp_hbm.at[cid, pl.ds(sid * _RPT, _RPT)])


_sc_prep = pl.kernel(
    _sc_prep_body,
    out_type=[jax.ShapeDtypeStruct((_NCHUNKS, _CHUNK), jnp.int32),
              jax.ShapeDtypeStruct((_NC, _NPAD), jnp.float32)],
    mesh=_MESH,
    scratch_types=[
        pltpu.VMEM((_CPW, _CHUNK), jnp.int32),
        pltpu.VMEM((_CPW, _CHUNK), jnp.int32),
        pltpu.VMEM((_CPW, _CHUNK), jnp.int32),
        pltpu.VMEM((_CPW, _CHUNK), jnp.int32),
        pltpu.VMEM((_CHUNK,), jnp.float32),
        pltpu.VMEM((_RPT,), jnp.float32),
        pltpu.VMEM_SHARED((_NPAD,), jnp.float32),
        pltpu.SemaphoreType.DMA,
    ],
)


def _sc_edges_body(table_hbm, gidx_hbm, dst_hbm, parts_hbm,
                   gidx_all, dst_all, rows0, rows1, rows2, rows3, acc,
                   gsem0, gsem1, gsem2, gsem3, ssem0, ssem1, ssem2, ssem3):
    cid = lax.axis_index("c")
    sid = lax.axis_index("s")
    wid = cid * _NS + sid

    pltpu.sync_copy(gidx_hbm.at[pl.ds(wid * _CPT * _ECH, _CPT * _ECH)],
                    gidx_all)
    pltpu.sync_copy(dst_hbm.at[pl.ds(wid * _CPT, _CPT)], dst_all)

    def zrow(r, carry):
        for k in range(_C // 16):
            rows0[r, pl.ds(k * 16, 16)] = jnp.zeros((16,), jnp.float32)
        return carry
    lax.fori_loop(0, _ECH, zrow, 0)
    for i in range(_RPT // _ECH):
        pltpu.sync_copy(rows0, acc.at[pl.ds(sid * _RPT + i * _ECH, _ECH)])
    plsc.subcore_barrier()

    rows = (rows0, rows1, rows2, rows3)
    gsem = (gsem0, gsem1, gsem2, gsem3)
    ssem = (ssem0, ssem1, ssem2, ssem3)
    def gsl(c):
        return gidx_all.at[pl.ds(c * _ECH, _ECH)]
    for b in range(4):
        pltpu.async_copy(table_hbm.at[gsl(b)], rows[b], gsem[b])

    def quad(q, carry):
        base = 4 * q
        for b in range(4):
            c = base + b
            pltpu.make_async_copy(
                table_hbm.at[gsl(c)], rows[b], gsem[b]).wait()
            pltpu.async_copy(rows[b], acc.at[dst_all.at[c]], ssem[b],
                             add=True)

            @pl.when(c + 4 < _CPT)
            def _():
                pltpu.make_async_copy(
                    rows[b], acc.at[dst_all.at[c]], ssem[b]).wait()
                pltpu.async_copy(
                    table_hbm.at[gsl(c + 4)], rows[b], gsem[b])
        return carry
    lax.fori_loop(0, _CPT // 4, quad, 0)
    for b in range(4):
        pltpu.make_async_copy(
            rows[b], acc.at[dst_all.at[_CPT - 4 + b]], ssem[b]).wait()
    plsc.subcore_barrier()
    pltpu.sync_copy(acc.at[pl.ds(sid * _RPT, _RPT)],
                    parts_hbm.at[cid, pl.ds(sid * _RPT, _RPT)])


_sc_edges = pl.kernel(
    _sc_edges_body,
    out_type=jax.ShapeDtypeStruct((_NC, _NPAD, _C), jnp.float32),
    mesh=_MESH,
    scratch_types=[
        pltpu.VMEM((_CPT * _ECH,), jnp.int32),
        pltpu.VMEM((_CPT, _ECH), jnp.int32),
        pltpu.VMEM((_ECH, _C), jnp.float32),
        pltpu.VMEM((_ECH, _C), jnp.float32),
        pltpu.VMEM((_ECH, _C), jnp.float32),
        pltpu.VMEM((_ECH, _C), jnp.float32),
        pltpu.VMEM_SHARED((_NPAD, _C), jnp.float32),
        pltpu.SemaphoreType.DMA,
        pltpu.SemaphoreType.DMA,
        pltpu.SemaphoreType.DMA,
        pltpu.SemaphoreType.DMA,
        pltpu.SemaphoreType.DMA,
        pltpu.SemaphoreType.DMA,
        pltpu.SemaphoreType.DMA,
        pltpu.SemaphoreType.DMA,
    ],
)


def _dinv_body(degp_ref, dinv_ref):
    d = degp_ref[0:1, :] + degp_ref[1:2, :]
    dinv_ref[...] = jnp.where(d > 0, lax.rsqrt(jnp.maximum(d, 1e-30)), 0.0)


_tc_dinv = pl.pallas_call(
    _dinv_body,
    out_shape=jax.ShapeDtypeStruct((1, _NPAD), jnp.float32),
)


def _lin1_body(x_ref, w_ref, b_ref, h_ref):
    h = jnp.dot(x_ref[...], w_ref[...], preferred_element_type=jnp.float32,
                precision=lax.Precision.HIGHEST)
    h_ref[...] = jnp.maximum(h + b_ref[...], 0.0)


_tc_lin1 = pl.pallas_call(
    _lin1_body,
    grid=(8,),
    in_specs=[
        pl.BlockSpec((1000, 256), lambda i: (i, 0)),
        pl.BlockSpec((256, _C), lambda i: (0, 0)),
        pl.BlockSpec((1, _C), lambda i: (0, 0)),
    ],
    out_specs=pl.BlockSpec((1000, _C), lambda i: (i, 0)),
    out_shape=jax.ShapeDtypeStruct((8000, _C), jnp.float32),
)


def _z_body(xin_ref, dinv_ref, w_ref, z_ref):
    xs = xin_ref[...] * dinv_ref[...]
    z_ref[...] = jnp.dot(xs, w_ref[0], preferred_element_type=jnp.float32,
                         precision=lax.Precision.HIGHEST)


_tc_z = pl.pallas_call(
    _z_body,
    grid=(_NREL, 5),
    in_specs=[
        pl.BlockSpec((2000, _C), lambda r, b: (b, 0)),
        pl.BlockSpec((2000, 1), lambda r, b: (b, 0)),
        pl.BlockSpec((1, _C, _C), lambda r, b: (r, 0, 0)),
    ],
    out_specs=pl.BlockSpec((2000, _C), lambda r, b: (r * 5 + b, 0)),
    out_shape=jax.ShapeDtypeStruct((_NREL * _N, _C), jnp.float32),
)


def _d_body(xin_ref, root_ref, b_ref, d_ref):
    d_ref[...] = jnp.dot(xin_ref[...], root_ref[...],
                         preferred_element_type=jnp.float32,
                         precision=lax.Precision.HIGHEST) + b_ref[...]


_tc_d = pl.pallas_call(
    _d_body,
    grid=(5,),
    in_specs=[
        pl.BlockSpec((2000, _C), lambda i: (i, 0)),
        pl.BlockSpec((_C, _C), lambda i: (0, 0)),
        pl.BlockSpec((1, _C), lambda i: (0, 0)),
    ],
    out_specs=pl.BlockSpec((2000, _C), lambda i: (i, 0)),
    out_shape=jax.ShapeDtypeStruct((_N, _C), jnp.float32),
)


_GROUPS = ((0, 8000), (8000, 8800), (8800, 9800), (9800, _N))


def _combine_body(parts_ref, d_ref, dinv_ref, out_ref, *, relu):
    agg = parts_ref[0][0:_N, :] + parts_ref[1][0:_N, :]
    u = d_ref[...] + dinv_ref[...] * agg
    if relu:
        u = jnp.maximum(u, 0.0)
    for a, b in _GROUPS:
        z = u[a:b, :]
        m = jnp.mean(z, axis=0, keepdims=True)
        v = jnp.mean((z - m) ** 2, axis=0, keepdims=True)
        out_ref[a:b, :] = (z - m) * lax.rsqrt(v + 1e-5)


def _make_combine(relu):
    return pl.pallas_call(
        functools.partial(_combine_body, relu=relu),
        out_shape=jax.ShapeDtypeStruct((_N, _C), jnp.float32),
    )


_tc_combine_relu = _make_combine(True)
_tc_combine_last = _make_combine(False)


def kernel(x, edge_index, edge_types, dis_emb, comp_emb, path_emb,
           lin1_W, lin1_b, root1, w1, b1, root2, w2, b2, root3, w3, b3):
    src = edge_index[0].astype(jnp.int32)
    dst = edge_index[1].astype(jnp.int32)
    et = edge_types.astype(jnp.int32)
    pad = _EPAD - _E
    ar = jnp.arange(pad, dtype=jnp.int32)
    src2 = jnp.concatenate([src, ar % _N]).reshape(_NCHUNKS, _CHUNK)
    et2 = jnp.concatenate([et, ar % _NREL]).reshape(_NCHUNKS, _CHUNK)
    dst2 = jnp.concatenate(
        [dst, _N + ar % (_NPAD - _N)]).reshape(_NCHUNKS, _CHUNK)

    gidx2, degp = _sc_prep(src2, et2, dst2)
    gidx_flat = gidx2.reshape(_EPAD)
    dst64 = dst2.reshape(_NCH2, _ECH)
    dinv_col = _tc_dinv(degp).reshape(_NPAD, 1)[:_N]

    h = _tc_lin1(x, lin1_W, lin1_b.reshape(1, _C))
    xin = jnp.concatenate([h, dis_emb, comp_emb, path_emb], axis=0)

    layers = ((root1, w1, b1, True), (root2, w2, b2, True),
              (root3, w3, b3, False))
    for root, w, b, relu in layers:
        z = _tc_z(xin, dinv_col, w)
        parts = _sc_edges(z, gidx_flat, dst64)
        d = _tc_d(xin, root, b.reshape(1, _C))
        if relu:
            xin = _tc_combine_relu(parts, d, dinv_col)
        else:
            xin = _tc_combine_last(parts, d, dinv_col)
    return xin


# speedup vs baseline: 1.1101x; 1.1101x over previous
import functools

import jax
import jax.numpy as jnp
from jax import lax
from jax.experimental import pallas as pl
from jax.experimental.pallas import tpu as pltpu
from jax.experimental.pallas import tpu_sc as plsc

_N = 10000
_NPAD = 10240
_E = 160000
_C = 128
_CHUNK = 128
_NCHUNKS = 1280
_EPAD = _NCHUNKS * _CHUNK
_NC, _NS = 2, 16
_NW = _NC * _NS
_CPW = _NCHUNKS // _NW
_ECH = 64
_NCH2 = _EPAD // _ECH
_CPT = _NCH2 // _NW
_RPT = _NPAD // _NS
_NREL = 3

_MESH = plsc.VectorSubcoreMesh(
    core_axis_name="c", subcore_axis_name="s", num_cores=_NC, num_subcores=_NS)


_NCR = _E // _CHUNK
_TAILR = _NCR - _CPW * (_NW - 1)


def _sc_prep_body(ei_hbm, et_hbm, gidx_hbm, dstp_hbm, degp_hbm,
                  src_all, et_all, dst_all, g_all, ones_v, zv, dacc, ssem):
    cid = lax.axis_index("c")
    sid = lax.axis_index("s")
    wid = cid * _NS + sid
    last = _NW - 1
    lo = wid * _CPW
    cnt = jnp.where(wid < last, _CPW, _TAILR)

    @pl.when(wid < last)
    def _():
        pltpu.sync_copy(ei_hbm.at[0, pl.ds(lo, _CPW)], src_all)
        pltpu.sync_copy(ei_hbm.at[1, pl.ds(lo, _CPW)], dst_all)
        pltpu.sync_copy(et_hbm.at[pl.ds(lo, _CPW)], et_all)

    @pl.when(wid == last)
    def _():
        pltpu.sync_copy(ei_hbm.at[0, pl.ds(last * _CPW, _TAILR)],
                        src_all.at[pl.ds(0, _TAILR)])
        pltpu.sync_copy(ei_hbm.at[1, pl.ds(last * _CPW, _TAILR)],
                        dst_all.at[pl.ds(0, _TAILR)])
        pltpu.sync_copy(et_hbm.at[pl.ds(last * _CPW, _TAILR)],
                        et_all.at[pl.ds(0, _TAILR)])

    def zstep(k, carry):
        zv[pl.ds(k * 16, 16)] = jnp.zeros((16,), jnp.float32)
        return carry
    lax.fori_loop(0, _RPT // 16, zstep, 0)
    for k in range(_CHUNK // 16):
        ones_v[pl.ds(k * 16, 16)] = jnp.ones((16,), jnp.float32)
    pltpu.sync_copy(zv, dacc.at[pl.ds(sid * _RPT, _RPT)])
    plsc.subcore_barrier()

    def step(t, carry):
        for k in range(_CHUNK // 16):
            sl = pl.ds(k * 16, 16)
            g_all[t, sl] = et_all[t, sl] * _N + src_all[t, sl]
        pltpu.async_copy(ones_v, dacc.at[dst_all.at[t]], ssem, add=True)
        return carry
    lax.fori_loop(0, cnt, step, 0)

    @pl.when(wid == last)
    def _():
        def padrow(r, carry):
            for k in range(_CHUNK // 16):
                sl = pl.ds(k * 16, 16)
                lane = lax.iota(jnp.int32, 16)
                v = r * 911 + k * 128 + lane * 8
                g_all[r, sl] = lax.rem(v, _NREL * _N)
                dst_all[r, sl] = _N + lax.rem(r * 128 + k * 16 + lane,
                                              _NPAD - _N)
            return carry
        lax.fori_loop(_TAILR, _CPW, padrow, 0)

    pltpu.sync_copy(g_all, gidx_hbm.at[pl.ds(lo, _CPW)])
    pltpu.sync_copy(dst_all, dstp_hbm.at[pl.ds(lo, _CPW)])

    def drain(t, carry):
        pltpu.make_async_copy(ones_v, dacc.at[dst_all.at[t]], ssem).wait()
        return carry
    lax.fori_loop(0, cnt, drain, 0)
    plsc.subcore_barrier()
    pltpu.sync_copy(dacc.at[pl.ds(sid * _RPT, _RPT)],
                    degp_hbm.at[cid, pl.ds(sid * _RPT, _RPT)])


_sc_prep = pl.kernel(
    _sc_prep_body,
    out_type=[jax.ShapeDtypeStruct((_NCHUNKS, _CHUNK), jnp.int32),
              jax.ShapeDtypeStruct((_NCHUNKS, _CHUNK), jnp.int32),
              jax.ShapeDtypeStruct((_NC, _NPAD), jnp.float32)],
    mesh=_MESH,
    scratch_types=[
        pltpu.VMEM((_CPW, _CHUNK), jnp.int32),
        pltpu.VMEM((_CPW, _CHUNK), jnp.int32),
        pltpu.VMEM((_CPW, _CHUNK), jnp.int32),
        pltpu.VMEM((_CPW, _CHUNK), jnp.int32),
        pltpu.VMEM((_CHUNK,), jnp.float32),
        pltpu.VMEM((_RPT,), jnp.float32),
        pltpu.VMEM_SHARED((_NPAD,), jnp.float32),
        pltpu.SemaphoreType.DMA,
    ],
)


def _sc_edges_body(table_hbm, gidx_hbm, dst_hbm, parts_hbm,
                   gidx_all, dst_all, rows0, rows1, rows2, rows3, acc,
                   gsem0, gsem1, gsem2, gsem3, ssem0, ssem1, ssem2, ssem3):
    cid = lax.axis_index("c")
    sid = lax.axis_index("s")
    wid = cid * _NS + sid

    pltpu.sync_copy(gidx_hbm.at[pl.ds(wid * _CPT * _ECH, _CPT * _ECH)],
                    gidx_all)
    pltpu.sync_copy(dst_hbm.at[pl.ds(wid * _CPT, _CPT)], dst_all)

    def zrow(r, carry):
        for k in range(_C // 16):
            rows0[r, pl.ds(k * 16, 16)] = jnp.zeros((16,), jnp.float32)
        return carry
    lax.fori_loop(0, _ECH, zrow, 0)
    for i in range(_RPT // _ECH):
        pltpu.sync_copy(rows0, acc.at[pl.ds(sid * _RPT + i * _ECH, _ECH)])
    plsc.subcore_barrier()

    rows = (rows0, rows1, rows2, rows3)
    gsem = (gsem0, gsem1, gsem2, gsem3)
    ssem = (ssem0, ssem1, ssem2, ssem3)
    def gsl(c):
        return gidx_all.at[pl.ds(c * _ECH, _ECH)]
    for b in range(4):
        pltpu.async_copy(table_hbm.at[gsl(b)], rows[b], gsem[b])

    def quad(q, carry):
        base = 4 * q
        for b in range(4):
            c = base + b
            pltpu.make_async_copy(
                table_hbm.at[gsl(c)], rows[b], gsem[b]).wait()
            pltpu.async_copy(rows[b], acc.at[dst_all.at[c]], ssem[b],
                             add=True)

            @pl.when(c + 4 < _CPT)
            def _():
                pltpu.make_async_copy(
                    rows[b], acc.at[dst_all.at[c]], ssem[b]).wait()
                pltpu.async_copy(
                    table_hbm.at[gsl(c + 4)], rows[b], gsem[b])
        return carry
    lax.fori_loop(0, _CPT // 4, quad, 0)
    for b in range(4):
        pltpu.make_async_copy(
            rows[b], acc.at[dst_all.at[_CPT - 4 + b]], ssem[b]).wait()
    plsc.subcore_barrier()
    pltpu.sync_copy(acc.at[pl.ds(sid * _RPT, _RPT)],
                    parts_hbm.at[cid, pl.ds(sid * _RPT, _RPT)])


_sc_edges = pl.kernel(
    _sc_edges_body,
    out_type=jax.ShapeDtypeStruct((_NC, _NPAD, _C), jnp.float32),
    mesh=_MESH,
    scratch_types=[
        pltpu.VMEM((_CPT * _ECH,), jnp.int32),
        pltpu.VMEM((_CPT, _ECH), jnp.int32),
        pltpu.VMEM((_ECH, _C), jnp.float32),
        pltpu.VMEM((_ECH, _C), jnp.float32),
        pltpu.VMEM((_ECH, _C), jnp.float32),
        pltpu.VMEM((_ECH, _C), jnp.float32),
        pltpu.VMEM_SHARED((_NPAD, _C), jnp.float32),
        pltpu.SemaphoreType.DMA,
        pltpu.SemaphoreType.DMA,
        pltpu.SemaphoreType.DMA,
        pltpu.SemaphoreType.DMA,
        pltpu.SemaphoreType.DMA,
        pltpu.SemaphoreType.DMA,
        pltpu.SemaphoreType.DMA,
        pltpu.SemaphoreType.DMA,
    ],
)


def _dinv_body(degp_ref, dinv_ref):
    d = degp_ref[0:1, :] + degp_ref[1:2, :]
    dinv_ref[...] = jnp.where(d > 0, lax.rsqrt(jnp.maximum(d, 1e-30)), 0.0)


_tc_dinv = pl.pallas_call(
    _dinv_body,
    out_shape=jax.ShapeDtypeStruct((1, _NPAD), jnp.float32),
)


def _lin1_body(x_ref, w_ref, b_ref, h_ref):
    h = jnp.dot(x_ref[...], w_ref[...], preferred_element_type=jnp.float32,
                precision=lax.Precision.HIGHEST)
    h_ref[...] = jnp.maximum(h + b_ref[...], 0.0)


_tc_lin1 = pl.pallas_call(
    _lin1_body,
    grid=(8,),
    in_specs=[
        pl.BlockSpec((1000, 256), lambda i: (i, 0)),
        pl.BlockSpec((256, _C), lambda i: (0, 0)),
        pl.BlockSpec((1, _C), lambda i: (0, 0)),
    ],
    out_specs=pl.BlockSpec((1000, _C), lambda i: (i, 0)),
    out_shape=jax.ShapeDtypeStruct((8000, _C), jnp.float32),
)


def _z_body(xin_ref, dinv_ref, w_ref, z_ref):
    xs = xin_ref[...] * dinv_ref[...]
    for i in range(_NREL):
        z_ref[i] = jnp.dot(xs, w_ref[i], preferred_element_type=jnp.float32,
                           precision=lax.Precision.HIGHEST)


_tc_z = pl.pallas_call(
    _z_body,
    grid=(5,),
    in_specs=[
        pl.BlockSpec((2000, _C), lambda b: (b, 0)),
        pl.BlockSpec((2000, 1), lambda b: (b, 0)),
        pl.BlockSpec((_NREL, _C, _C), lambda b: (0, 0, 0)),
    ],
    out_specs=pl.BlockSpec((_NREL, 2000, _C), lambda b: (0, b, 0)),
    out_shape=jax.ShapeDtypeStruct((_NREL, _N, _C), jnp.float32),
)


def _d_body(xin_ref, root_ref, b_ref, d_ref):
    d_ref[...] = jnp.dot(xin_ref[...], root_ref[...],
                         preferred_element_type=jnp.float32,
                         precision=lax.Precision.HIGHEST) + b_ref[...]


_tc_d = pl.pallas_call(
    _d_body,
    grid=(5,),
    in_specs=[
        pl.BlockSpec((2000, _C), lambda i: (i, 0)),
        pl.BlockSpec((_C, _C), lambda i: (0, 0)),
        pl.BlockSpec((1, _C), lambda i: (0, 0)),
    ],
    out_specs=pl.BlockSpec((2000, _C), lambda i: (i, 0)),
    out_shape=jax.ShapeDtypeStruct((_N, _C), jnp.float32),
)


_GROUPS = ((0, 8000), (8000, 8800), (8800, 9800), (9800, _N))


def _combine_body(parts_ref, d_ref, dinv_ref, out_ref, *, relu):
    agg = parts_ref[0][0:_N, :] + parts_ref[1][0:_N, :]
    u = d_ref[...] + dinv_ref[...] * agg
    if relu:
        u = jnp.maximum(u, 0.0)
    for a, b in _GROUPS:
        z = u[a:b, :]
        m = jnp.mean(z, axis=0, keepdims=True)
        v = jnp.mean((z - m) ** 2, axis=0, keepdims=True)
        out_ref[a:b, :] = (z - m) * lax.rsqrt(v + 1e-5)


def _make_combine(relu):
    return pl.pallas_call(
        functools.partial(_combine_body, relu=relu),
        out_shape=jax.ShapeDtypeStruct((_N, _C), jnp.float32),
    )


_tc_combine_relu = _make_combine(True)
_tc_combine_last = _make_combine(False)


def kernel(x, edge_index, edge_types, dis_emb, comp_emb, path_emb,
           lin1_W, lin1_b, root1, w1, b1, root2, w2, b2, root3, w3, b3):
    ei3 = edge_index.astype(jnp.int32).reshape(2, _NCR, _CHUNK)
    et3 = edge_types.astype(jnp.int32).reshape(_NCR, _CHUNK)

    gidx2, dstp, degp = _sc_prep(ei3, et3)
    gidx_flat = gidx2.reshape(_EPAD)
    dst64 = dstp.reshape(_NCH2, _ECH)
    dinv_col = _tc_dinv(degp).reshape(_NPAD, 1)[:_N]

    h = _tc_lin1(x, lin1_W, lin1_b.reshape(1, _C))
    xin = jnp.concatenate([h, dis_emb, comp_emb, path_emb], axis=0)

    layers = ((root1, w1, b1, True), (root2, w2, b2, True),
              (root3, w3, b3, False))
    for root, w, b, relu in layers:
        z = _tc_z(xin, dinv_col, w)
        parts = _sc_edges(z.reshape(_NREL * _N, _C), gidx_flat, dst64)
        d = _tc_d(xin, root, b.reshape(1, _C))
        if relu:
            xin = _tc_combine_relu(parts, d, dinv_col)
        else:
            xin = _tc_combine_last(parts, d, dinv_col)
    return xin


# speedup vs baseline: 1.1367x; 1.0240x over previous
import functools

import jax
import jax.numpy as jnp
from jax import lax
from jax.experimental import pallas as pl
from jax.experimental.pallas import tpu as pltpu
from jax.experimental.pallas import tpu_sc as plsc

_N = 10000
_NPAD = 10240
_E = 160000
_C = 128
_CHUNK = 128
_NCHUNKS = 1280
_EPAD = _NCHUNKS * _CHUNK
_NC, _NS = 2, 16
_NW = _NC * _NS
_CPW = _NCHUNKS // _NW
_ECH = 64
_NCH2 = _EPAD // _ECH
_CPT = _NCH2 // _NW
_RPT = _NPAD // _NS
_NREL = 3

_MESH = plsc.VectorSubcoreMesh(
    core_axis_name="c", subcore_axis_name="s", num_cores=_NC, num_subcores=_NS)


_NCR = _E // _CHUNK
_TAILR = _NCR - _CPW * (_NW - 1)


def _sc_prep_body(ei_hbm, et_hbm, gidx_hbm, dstp_hbm, degp_hbm,
                  src_all, et_all, dst_all, g_all, ones_v, zv, dacc, ssem):
    cid = lax.axis_index("c")
    sid = lax.axis_index("s")
    wid = cid * _NS + sid
    last = _NW - 1
    lo = wid * _CPW
    cnt = jnp.where(wid < last, _CPW, _TAILR)

    @pl.when(wid < last)
    def _():
        pltpu.sync_copy(ei_hbm.at[0, pl.ds(lo, _CPW)], src_all)
        pltpu.sync_copy(ei_hbm.at[1, pl.ds(lo, _CPW)], dst_all)
        pltpu.sync_copy(et_hbm.at[pl.ds(lo, _CPW)], et_all)

    @pl.when(wid == last)
    def _():
        pltpu.sync_copy(ei_hbm.at[0, pl.ds(last * _CPW, _TAILR)],
                        src_all.at[pl.ds(0, _TAILR)])
        pltpu.sync_copy(ei_hbm.at[1, pl.ds(last * _CPW, _TAILR)],
                        dst_all.at[pl.ds(0, _TAILR)])
        pltpu.sync_copy(et_hbm.at[pl.ds(last * _CPW, _TAILR)],
                        et_all.at[pl.ds(0, _TAILR)])

    def zstep(k, carry):
        zv[pl.ds(k * 16, 16)] = jnp.zeros((16,), jnp.float32)
        return carry
    lax.fori_loop(0, _RPT // 16, zstep, 0)
    for k in range(_CHUNK // 16):
        ones_v[pl.ds(k * 16, 16)] = jnp.ones((16,), jnp.float32)
    pltpu.sync_copy(zv, dacc.at[pl.ds(sid * _RPT, _RPT)])
    plsc.subcore_barrier()

    def step(t, carry):
        for k in range(_CHUNK // 16):
            sl = pl.ds(k * 16, 16)
            g_all[t, sl] = et_all[t, sl] * _N + src_all[t, sl]
        pltpu.async_copy(ones_v, dacc.at[dst_all.at[t]], ssem, add=True)
        return carry
    lax.fori_loop(0, cnt, step, 0)

    @pl.when(wid == last)
    def _():
        def padrow(r, carry):
            for k in range(_CHUNK // 16):
                sl = pl.ds(k * 16, 16)
                lane = lax.iota(jnp.int32, 16)
                v = r * 911 + k * 128 + lane * 8
                g_all[r, sl] = lax.rem(v, _NREL * _N)
                dst_all[r, sl] = _N + lax.rem(r * 128 + k * 16 + lane,
                                              _NPAD - _N)
            return carry
        lax.fori_loop(_TAILR, _CPW, padrow, 0)

    pltpu.sync_copy(g_all, gidx_hbm.at[pl.ds(lo, _CPW)])
    pltpu.sync_copy(dst_all, dstp_hbm.at[pl.ds(lo, _CPW)])

    def drain(t, carry):
        pltpu.make_async_copy(ones_v, dacc.at[dst_all.at[t]], ssem).wait()
        return carry
    lax.fori_loop(0, cnt, drain, 0)
    plsc.subcore_barrier()
    pltpu.sync_copy(dacc.at[pl.ds(sid * _RPT, _RPT)],
                    degp_hbm.at[cid, pl.ds(sid * _RPT, _RPT)])


_sc_prep = pl.kernel(
    _sc_prep_body,
    out_type=[jax.ShapeDtypeStruct((_NCHUNKS, _CHUNK), jnp.int32),
              jax.ShapeDtypeStruct((_NCHUNKS, _CHUNK), jnp.int32),
              jax.ShapeDtypeStruct((_NC, _NPAD), jnp.float32)],
    mesh=_MESH,
    scratch_types=[
        pltpu.VMEM((_CPW, _CHUNK), jnp.int32),
        pltpu.VMEM((_CPW, _CHUNK), jnp.int32),
        pltpu.VMEM((_CPW, _CHUNK), jnp.int32),
        pltpu.VMEM((_CPW, _CHUNK), jnp.int32),
        pltpu.VMEM((_CHUNK,), jnp.float32),
        pltpu.VMEM((_RPT,), jnp.float32),
        pltpu.VMEM_SHARED((_NPAD,), jnp.float32),
        pltpu.SemaphoreType.DMA,
    ],
)


def _sc_edges_body(table_hbm, gidx_hbm, dst_hbm, parts_hbm,
                   gidx_all, dst_all, rows0, rows1, rows2, rows3, acc,
                   gsem0, gsem1, gsem2, gsem3, ssem0, ssem1, ssem2, ssem3):
    cid = lax.axis_index("c")
    sid = lax.axis_index("s")
    wid = cid * _NS + sid

    pltpu.async_copy(gidx_hbm.at[pl.ds(wid * _CPT * _ECH, _CPT * _ECH)],
                     gidx_all, ssem0)
    pltpu.async_copy(dst_hbm.at[pl.ds(wid * _CPT, _CPT)], dst_all, ssem1)

    def zrow(r, carry):
        for k in range(_C // 16):
            rows0[r, pl.ds(k * 16, 16)] = jnp.zeros((16,), jnp.float32)
        return carry
    lax.fori_loop(0, _ECH, zrow, 0)
    for i in range(_RPT // _ECH):
        pltpu.async_copy(rows0, acc.at[pl.ds(sid * _RPT + i * _ECH, _ECH)],
                         gsem0)
    for i in range(_RPT // _ECH):
        pltpu.make_async_copy(
            rows0, acc.at[pl.ds(sid * _RPT + i * _ECH, _ECH)], gsem0).wait()
    pltpu.make_async_copy(
        gidx_hbm.at[pl.ds(wid * _CPT * _ECH, _CPT * _ECH)], gidx_all,
        ssem0).wait()
    pltpu.make_async_copy(
        dst_hbm.at[pl.ds(wid * _CPT, _CPT)], dst_all, ssem1).wait()
    plsc.subcore_barrier()

    rows = (rows0, rows1, rows2, rows3)
    gsem = (gsem0, gsem1, gsem2, gsem3)
    ssem = (ssem0, ssem1, ssem2, ssem3)
    def gsl(c):
        return gidx_all.at[pl.ds(c * _ECH, _ECH)]
    for b in range(4):
        pltpu.async_copy(table_hbm.at[gsl(b)], rows[b], gsem[b])

    def quad(q, carry):
        base = 4 * q
        for b in range(4):
            c = base + b
            pltpu.make_async_copy(
                table_hbm.at[gsl(c)], rows[b], gsem[b]).wait()
            pltpu.async_copy(rows[b], acc.at[dst_all.at[c]], ssem[b],
                             add=True)

            @pl.when(c + 4 < _CPT)
            def _():
                pltpu.make_async_copy(
                    rows[b], acc.at[dst_all.at[c]], ssem[b]).wait()
                pltpu.async_copy(
                    table_hbm.at[gsl(c + 4)], rows[b], gsem[b])
        return carry
    lax.fori_loop(0, _CPT // 4, quad, 0)
    for b in range(4):
        pltpu.make_async_copy(
            rows[b], acc.at[dst_all.at[_CPT - 4 + b]], ssem[b]).wait()
    plsc.subcore_barrier()
    pltpu.sync_copy(acc.at[pl.ds(sid * _RPT, _RPT)],
                    parts_hbm.at[cid, pl.ds(sid * _RPT, _RPT)])


_sc_edges = pl.kernel(
    _sc_edges_body,
    out_type=jax.ShapeDtypeStruct((_NC, _NPAD, _C), jnp.float32),
    mesh=_MESH,
    scratch_types=[
        pltpu.VMEM((_CPT * _ECH,), jnp.int32),
        pltpu.VMEM((_CPT, _ECH), jnp.int32),
        pltpu.VMEM((_ECH, _C), jnp.float32),
        pltpu.VMEM((_ECH, _C), jnp.float32),
        pltpu.VMEM((_ECH, _C), jnp.float32),
        pltpu.VMEM((_ECH, _C), jnp.float32),
        pltpu.VMEM_SHARED((_NPAD, _C), jnp.float32),
        pltpu.SemaphoreType.DMA,
        pltpu.SemaphoreType.DMA,
        pltpu.SemaphoreType.DMA,
        pltpu.SemaphoreType.DMA,
        pltpu.SemaphoreType.DMA,
        pltpu.SemaphoreType.DMA,
        pltpu.SemaphoreType.DMA,
        pltpu.SemaphoreType.DMA,
    ],
)


def _dinv_body(degp_ref, dinv_ref):
    d = degp_ref[0:1, :] + degp_ref[1:2, :]
    dinv_ref[...] = jnp.where(d > 0, lax.rsqrt(jnp.maximum(d, 1e-30)), 0.0)


_tc_dinv = pl.pallas_call(
    _dinv_body,
    out_shape=jax.ShapeDtypeStruct((1, _NPAD), jnp.float32),
)


def _lin1_body(x_ref, w_ref, b_ref, h_ref):
    h = jnp.dot(x_ref[...], w_ref[...], preferred_element_type=jnp.float32,
                precision=lax.Precision.HIGHEST)
    h_ref[...] = jnp.maximum(h + b_ref[...], 0.0)


_tc_lin1 = pl.pallas_call(
    _lin1_body,
    grid=(8,),
    in_specs=[
        pl.BlockSpec((1000, 256), lambda i: (i, 0)),
        pl.BlockSpec((256, _C), lambda i: (0, 0)),
        pl.BlockSpec((1, _C), lambda i: (0, 0)),
    ],
    out_specs=pl.BlockSpec((1000, _C), lambda i: (i, 0)),
    out_shape=jax.ShapeDtypeStruct((8000, _C), jnp.float32),
)


def _z_body(xin_ref, dinv_ref, w_ref, z_ref):
    xs = xin_ref[...] * dinv_ref[...]
    for i in range(_NREL):
        z_ref[i] = jnp.dot(xs, w_ref[i], preferred_element_type=jnp.float32,
                           precision=lax.Precision.HIGHEST)


_tc_z = pl.pallas_call(
    _z_body,
    grid=(5,),
    in_specs=[
        pl.BlockSpec((2000, _C), lambda b: (b, 0)),
        pl.BlockSpec((2000, 1), lambda b: (b, 0)),
        pl.BlockSpec((_NREL, _C, _C), lambda b: (0, 0, 0)),
    ],
    out_specs=pl.BlockSpec((_NREL, 2000, _C), lambda b: (0, b, 0)),
    out_shape=jax.ShapeDtypeStruct((_NREL, _N, _C), jnp.float32),
)


def _d_body(xin_ref, root_ref, b_ref, d_ref):
    d_ref[...] = jnp.dot(xin_ref[...], root_ref[...],
                         preferred_element_type=jnp.float32,
                         precision=lax.Precision.HIGHEST) + b_ref[...]


_tc_d = pl.pallas_call(
    _d_body,
    grid=(5,),
    in_specs=[
        pl.BlockSpec((2000, _C), lambda i: (i, 0)),
        pl.BlockSpec((_C, _C), lambda i: (0, 0)),
        pl.BlockSpec((1, _C), lambda i: (0, 0)),
    ],
    out_specs=pl.BlockSpec((2000, _C), lambda i: (i, 0)),
    out_shape=jax.ShapeDtypeStruct((_N, _C), jnp.float32),
)


_GROUPS = ((0, 8000), (8000, 8800), (8800, 9800), (9800, _N))


def _combine_body(parts_ref, d_ref, dinv_ref, out_ref, *, relu):
    agg = parts_ref[0][0:_N, :] + parts_ref[1][0:_N, :]
    u = d_ref[...] + dinv_ref[...] * agg
    if relu:
        u = jnp.maximum(u, 0.0)
    for a, b in _GROUPS:
        z = u[a:b, :]
        m = jnp.mean(z, axis=0, keepdims=True)
        v = jnp.mean((z - m) ** 2, axis=0, keepdims=True)
        out_ref[a:b, :] = (z - m) * lax.rsqrt(v + 1e-5)


def _make_combine(relu):
    return pl.pallas_call(
        functools.partial(_combine_body, relu=relu),
        out_shape=jax.ShapeDtypeStruct((_N, _C), jnp.float32),
    )


_tc_combine_relu = _make_combine(True)
_tc_combine_last = _make_combine(False)


def kernel(x, edge_index, edge_types, dis_emb, comp_emb, path_emb,
           lin1_W, lin1_b, root1, w1, b1, root2, w2, b2, root3, w3, b3):
    ei3 = edge_index.astype(jnp.int32).reshape(2, _NCR, _CHUNK)
    et3 = edge_types.astype(jnp.int32).reshape(_NCR, _CHUNK)

    gidx2, dstp, degp = _sc_prep(ei3, et3)
    gidx_flat = gidx2.reshape(_EPAD)
    dst64 = dstp.reshape(_NCH2, _ECH)
    dinv_col = _tc_dinv(degp).reshape(_NPAD, 1)[:_N]

    h = _tc_lin1(x, lin1_W, lin1_b.reshape(1, _C))
    xin = jnp.concatenate([h, dis_emb, comp_emb, path_emb], axis=0)

    layers = ((root1, w1, b1, True), (root2, w2, b2, True),
              (root3, w3, b3, False))
    for root, w, b, relu in layers:
        z = _tc_z(xin, dinv_col, w)
        parts = _sc_edges(z.reshape(_NREL * _N, _C), gidx_flat, dst64)
        d = _tc_d(xin, root, b.reshape(1, _C))
        if relu:
            xin = _tc_combine_relu(parts, d, dinv_col)
        else:
            xin = _tc_combine_last(parts, d, dinv_col)
    return xin


# speedup vs baseline: 1.1457x; 1.0080x over previous
import functools

import jax
import jax.numpy as jnp
from jax import lax
from jax.experimental import pallas as pl
from jax.experimental.pallas import tpu as pltpu
from jax.experimental.pallas import tpu_sc as plsc

_N = 10000
_NPAD = 10240
_E = 160000
_C = 128
_CHUNK = 128
_NCHUNKS = 1280
_EPAD = _NCHUNKS * _CHUNK
_NC, _NS = 2, 16
_NW = _NC * _NS
_CPW = _NCHUNKS // _NW
_ECH = 64
_NCH2 = _EPAD // _ECH
_CPT = _NCH2 // _NW
_RPT = _NPAD // _NS
_NREL = 3

_MESH = plsc.VectorSubcoreMesh(
    core_axis_name="c", subcore_axis_name="s", num_cores=_NC, num_subcores=_NS)


_NCR = _E // _CHUNK
_TAILR = _NCR - _CPW * (_NW - 1)
_EPW = _CPW * _CHUNK
_TAILE = _TAILR * _CHUNK
_DPW = _EPW // _ECH


def _sc_prep_body(ei_hbm, et_hbm, gidx_hbm, dstp_hbm, degp_hbm,
                  src_all, et_all, dst1, dst_all, g_all, ones_v, zv, dacc,
                  ssem):
    cid = lax.axis_index("c")
    sid = lax.axis_index("s")
    wid = cid * _NS + sid
    last = _NW - 1
    lo = wid * _EPW
    cnt2 = jnp.where(wid < last, _DPW, 2 * _TAILR)

    @pl.when(wid < last)
    def _():
        pltpu.sync_copy(ei_hbm.at[0, pl.ds(lo, _EPW)], src_all)
        pltpu.sync_copy(ei_hbm.at[1, pl.ds(lo, _EPW)], dst1)
        pltpu.sync_copy(et_hbm.at[pl.ds(lo, _EPW)], et_all)

    @pl.when(wid == last)
    def _():
        pltpu.sync_copy(ei_hbm.at[0, pl.ds(last * _EPW, _TAILE)],
                        src_all.at[pl.ds(0, _TAILE)])
        pltpu.sync_copy(ei_hbm.at[1, pl.ds(last * _EPW, _TAILE)],
                        dst1.at[pl.ds(0, _TAILE)])
        pltpu.sync_copy(et_hbm.at[pl.ds(last * _EPW, _TAILE)],
                        et_all.at[pl.ds(0, _TAILE)])

    def zstep(k, carry):
        zv[pl.ds(k * 16, 16)] = jnp.zeros((16,), jnp.float32)
        return carry
    lax.fori_loop(0, _RPT // 16, zstep, 0)
    for k in range(_ECH // 16):
        ones_v[pl.ds(k * 16, 16)] = jnp.ones((16,), jnp.float32)
    pltpu.sync_copy(zv, dacc.at[pl.ds(sid * _RPT, _RPT)])
    plsc.subcore_barrier()

    def step(j, carry):
        for k in range(_ECH // 16):
            sl = pl.ds(j * _ECH + k * 16, 16)
            g_all[sl] = et_all[sl] * _N + src_all[sl]
            dst_all[j, pl.ds(k * 16, 16)] = dst1[sl]
        pltpu.async_copy(ones_v, dacc.at[dst_all.at[j]], ssem, add=True)
        return carry
    lax.fori_loop(0, cnt2, step, 0)

    @pl.when(wid == last)
    def _():
        def padrow(j, carry):
            for k in range(_ECH // 16):
                sl = pl.ds(k * 16, 16)
                lane = lax.iota(jnp.int32, 16)
                v = j * 911 + k * 128 + lane * 8
                g_all[pl.ds(j * _ECH + k * 16, 16)] = lax.rem(v, _NREL * _N)
                dst_all[j, sl] = _N + lax.rem(j * 64 + k * 16 + lane,
                                              _NPAD - _N)
            return carry
        lax.fori_loop(2 * _TAILR, _DPW, padrow, 0)

    pltpu.sync_copy(g_all, gidx_hbm.at[pl.ds(lo, _EPW)])
    pltpu.sync_copy(dst_all, dstp_hbm.at[pl.ds(wid * _DPW, _DPW)])

    def drain(t, carry):
        pltpu.make_async_copy(ones_v, dacc.at[dst_all.at[t]], ssem).wait()
        return carry
    lax.fori_loop(0, cnt2, drain, 0)
    plsc.subcore_barrier()
    pltpu.sync_copy(dacc.at[pl.ds(sid * _RPT, _RPT)],
                    degp_hbm.at[cid, pl.ds(sid * _RPT, _RPT)])


_sc_prep = pl.kernel(
    _sc_prep_body,
    out_type=[jax.ShapeDtypeStruct((_EPAD,), jnp.int32),
              jax.ShapeDtypeStruct((_NCH2, _ECH), jnp.int32),
              jax.ShapeDtypeStruct((_NC, _NPAD), jnp.float32)],
    mesh=_MESH,
    scratch_types=[
        pltpu.VMEM((_EPW,), jnp.int32),
        pltpu.VMEM((_EPW,), jnp.int32),
        pltpu.VMEM((_EPW,), jnp.int32),
        pltpu.VMEM((_DPW, _ECH), jnp.int32),
        pltpu.VMEM((_EPW,), jnp.int32),
        pltpu.VMEM((_ECH,), jnp.float32),
        pltpu.VMEM((_RPT,), jnp.float32),
        pltpu.VMEM_SHARED((_NPAD,), jnp.float32),
        pltpu.SemaphoreType.DMA,
    ],
)


def _sc_edges_body(table_hbm, gidx_hbm, dst_hbm, parts_hbm,
                   gidx_all, dst_all, rows0, rows1, rows2, rows3, acc,
                   gsem0, gsem1, gsem2, gsem3, ssem0, ssem1, ssem2, ssem3):
    cid = lax.axis_index("c")
    sid = lax.axis_index("s")
    wid = cid * _NS + sid

    pltpu.async_copy(gidx_hbm.at[pl.ds(wid * _CPT * _ECH, _CPT * _ECH)],
                     gidx_all, ssem0)
    pltpu.async_copy(dst_hbm.at[pl.ds(wid * _CPT, _CPT)], dst_all, ssem1)

    def zrow(r, carry):
        for k in range(_C // 16):
            rows0[r, pl.ds(k * 16, 16)] = jnp.zeros((16,), jnp.float32)
        return carry
    lax.fori_loop(0, _ECH, zrow, 0)
    for i in range(_RPT // _ECH):
        pltpu.async_copy(rows0, acc.at[pl.ds(sid * _RPT + i * _ECH, _ECH)],
                         gsem0)
    for i in range(_RPT // _ECH):
        pltpu.make_async_copy(
            rows0, acc.at[pl.ds(sid * _RPT + i * _ECH, _ECH)], gsem0).wait()
    pltpu.make_async_copy(
        gidx_hbm.at[pl.ds(wid * _CPT * _ECH, _CPT * _ECH)], gidx_all,
        ssem0).wait()
    pltpu.make_async_copy(
        dst_hbm.at[pl.ds(wid * _CPT, _CPT)], dst_all, ssem1).wait()
    plsc.subcore_barrier()

    rows = (rows0, rows1, rows2, rows3)
    gsem = (gsem0, gsem1, gsem2, gsem3)
    ssem = (ssem0, ssem1, ssem2, ssem3)
    def gsl(c):
        return gidx_all.at[pl.ds(c * _ECH, _ECH)]
    for b in range(4):
        pltpu.async_copy(table_hbm.at[gsl(b)], rows[b], gsem[b])

    def quad(q, carry):
        base = 4 * q
        for b in range(4):
            c = base + b
            pltpu.make_async_copy(
                table_hbm.at[gsl(c)], rows[b], gsem[b]).wait()
            pltpu.async_copy(rows[b], acc.at[dst_all.at[c]], ssem[b],
                             add=True)

            @pl.when(c + 4 < _CPT)
            def _():
                pltpu.make_async_copy(
                    rows[b], acc.at[dst_all.at[c]], ssem[b]).wait()
                pltpu.async_copy(
                    table_hbm.at[gsl(c + 4)], rows[b], gsem[b])
        return carry
    lax.fori_loop(0, _CPT // 4, quad, 0)
    for b in range(4):
        pltpu.make_async_copy(
            rows[b], acc.at[dst_all.at[_CPT - 4 + b]], ssem[b]).wait()
    plsc.subcore_barrier()
    pltpu.sync_copy(acc.at[pl.ds(sid * _RPT, _RPT)],
                    parts_hbm.at[cid, pl.ds(sid * _RPT, _RPT)])


_sc_edges = pl.kernel(
    _sc_edges_body,
    out_type=jax.ShapeDtypeStruct((_NC, _NPAD, _C), jnp.float32),
    mesh=_MESH,
    scratch_types=[
        pltpu.VMEM((_CPT * _ECH,), jnp.int32),
        pltpu.VMEM((_CPT, _ECH), jnp.int32),
        pltpu.VMEM((_ECH, _C), jnp.float32),
        pltpu.VMEM((_ECH, _C), jnp.float32),
        pltpu.VMEM((_ECH, _C), jnp.float32),
        pltpu.VMEM((_ECH, _C), jnp.float32),
        pltpu.VMEM_SHARED((_NPAD, _C), jnp.float32),
        pltpu.SemaphoreType.DMA,
        pltpu.SemaphoreType.DMA,
        pltpu.SemaphoreType.DMA,
        pltpu.SemaphoreType.DMA,
        pltpu.SemaphoreType.DMA,
        pltpu.SemaphoreType.DMA,
        pltpu.SemaphoreType.DMA,
        pltpu.SemaphoreType.DMA,
    ],
)


def _dinv_body(degp_ref, dinv_ref):
    d = degp_ref[0:1, :] + degp_ref[1:2, :]
    r = jnp.where(d > 0, lax.rsqrt(jnp.maximum(d, 1e-30)), 0.0)
    dinv_ref[...] = jnp.reshape(r, (_NPAD, 1))


_tc_dinv = pl.pallas_call(
    _dinv_body,
    out_shape=jax.ShapeDtypeStruct((_NPAD, 1), jnp.float32),
)


def _lin1_body(x_ref, w_ref, b_ref, h_ref):
    h = jnp.dot(x_ref[...], w_ref[...], preferred_element_type=jnp.float32,
                precision=lax.Precision.HIGHEST)
    h_ref[...] = jnp.maximum(h + b_ref[...], 0.0)


_tc_lin1 = pl.pallas_call(
    _lin1_body,
    grid=(8,),
    in_specs=[
        pl.BlockSpec((1000, 256), lambda i: (i, 0)),
        pl.BlockSpec((256, _C), lambda i: (0, 0)),
        pl.BlockSpec((1, _C), lambda i: (0, 0)),
    ],
    out_specs=pl.BlockSpec((1000, _C), lambda i: (i, 0)),
    out_shape=jax.ShapeDtypeStruct((8000, _C), jnp.float32),
)


def _z_body(xin_ref, dinv_ref, w_ref, z_ref):
    xs = xin_ref[...] * dinv_ref[...]
    for i in range(_NREL):
        z_ref[i] = jnp.dot(xs, w_ref[i], preferred_element_type=jnp.float32,
                           precision=lax.Precision.HIGHEST)


_tc_z = pl.pallas_call(
    _z_body,
    grid=(5,),
    in_specs=[
        pl.BlockSpec((2000, _C), lambda b: (b, 0)),
        pl.BlockSpec((2000, 1), lambda b: (b, 0)),
        pl.BlockSpec((_NREL, _C, _C), lambda b: (0, 0, 0)),
    ],
    out_specs=pl.BlockSpec((_NREL, 2000, _C), lambda b: (0, b, 0)),
    out_shape=jax.ShapeDtypeStruct((_NREL, _N, _C), jnp.float32),
)


def _d_body(xin_ref, root_ref, b_ref, d_ref):
    d_ref[...] = jnp.dot(xin_ref[...], root_ref[...],
                         preferred_element_type=jnp.float32,
                         precision=lax.Precision.HIGHEST) + b_ref[...]


_tc_d = pl.pallas_call(
    _d_body,
    grid=(5,),
    in_specs=[
        pl.BlockSpec((2000, _C), lambda i: (i, 0)),
        pl.BlockSpec((_C, _C), lambda i: (0, 0)),
        pl.BlockSpec((1, _C), lambda i: (0, 0)),
    ],
    out_specs=pl.BlockSpec((2000, _C), lambda i: (i, 0)),
    out_shape=jax.ShapeDtypeStruct((_N, _C), jnp.float32),
)


_GROUPS = ((0, 8000), (8000, 8800), (8800, 9800), (9800, _N))


def _combine_body(parts_ref, d_ref, dinv_ref, out_ref, *, relu):
    agg = parts_ref[0][0:_N, :] + parts_ref[1][0:_N, :]
    u = d_ref[...] + dinv_ref[...] * agg
    if relu:
        u = jnp.maximum(u, 0.0)
    for a, b in _GROUPS:
        z = u[a:b, :]
        m = jnp.mean(z, axis=0, keepdims=True)
        v = jnp.mean((z - m) ** 2, axis=0, keepdims=True)
        out_ref[a:b, :] = (z - m) * lax.rsqrt(v + 1e-5)


def _make_combine(relu):
    return pl.pallas_call(
        functools.partial(_combine_body, relu=relu),
        out_shape=jax.ShapeDtypeStruct((_N, _C), jnp.float32),
    )


_tc_combine_relu = _make_combine(True)
_tc_combine_last = _make_combine(False)


def kernel(x, edge_index, edge_types, dis_emb, comp_emb, path_emb,
           lin1_W, lin1_b, root1, w1, b1, root2, w2, b2, root3, w3, b3):
    ei = edge_index.astype(jnp.int32)
    et1 = edge_types.astype(jnp.int32)

    gidx_flat, dst64, degp = _sc_prep(ei, et1)
    dinv_col = _tc_dinv(degp)[:_N]

    h = _tc_lin1(x, lin1_W, lin1_b.reshape(1, _C))
    xin = jnp.concatenate([h, dis_emb, comp_emb, path_emb], axis=0)

    layers = ((root1, w1, b1, True), (root2, w2, b2, True),
              (root3, w3, b3, False))
    for root, w, b, relu in layers:
        z = _tc_z(xin, dinv_col, w)
        parts = _sc_edges(z.reshape(_NREL * _N, _C), gidx_flat, dst64)
        d = _tc_d(xin, root, b.reshape(1, _C))
        if relu:
            xin = _tc_combine_relu(parts, d, dinv_col)
        else:
            xin = _tc_combine_last(parts, d, dinv_col)
    return xin
